# baseline (speedup 1.0000x reference)
import functools

import jax
import jax.numpy as jnp
from jax import lax
from jax.experimental import pallas as pl
from jax.experimental.pallas import tpu as pltpu
from jax.experimental.pallas import tpu_sc as plsc

VOCAB = 1000000
EMBED = 32
BATCH = 16384
NC, NS, L = 2, 16, 16
NW = NC * NS
BPW = BATCH // NW
CH = 128
NCH = BPW // CH

NGRP = 4
NT = 7813
GRP_WORDS = NT * 1024
TOTAL_TILES = NGRP * NT
WIN = 16
NWIN = (NT + WIN - 1) // WIN

_mesh = plsc.VectorSubcoreMesh(core_axis_name="c", subcore_axis_name="s")


def _tc_retile_body(c_in, o_in, c_out, o_out):
    for i in range(WIN):
        sl = pl.ds(i * 128, 128)
        c_out[0, i] = c_in[:, sl]
        o_out[0, i] = o_in[:, sl]


_tc_retile = pl.pallas_call(
    _tc_retile_body,
    grid=(NGRP, NWIN),
    in_specs=[
        pl.BlockSpec((8, WIN * 128), lambda g, w: (g, w)),
        pl.BlockSpec((8, WIN * 128), lambda g, w: (g, w)),
    ],
    out_specs=[
        pl.BlockSpec((1, WIN, 8, 128), lambda g, w: (g, w, 0, 0)),
        pl.BlockSpec((1, WIN, 8, 128), lambda g, w: (g, w, 0, 0)),
    ],
    out_shape=[
        jax.ShapeDtypeStruct((NGRP, NT, 8, 128), jnp.float32),
        jax.ShapeDtypeStruct((NGRP, NT, 8, 128), jnp.float32),
    ],
)


@functools.partial(
    pl.kernel,
    out_type=jax.ShapeDtypeStruct((BATCH,), jnp.float32),
    mesh=_mesh,
    compiler_params=pltpu.CompilerParams(
        use_tc_tiling_on_sc=False, needs_layout_passes=False),
    scratch_types=[
        pltpu.VMEM((NCH, CH), jnp.int32),
        pltpu.VMEM((NCH, CH), jnp.int32),
        pltpu.VMEM((EMBED, BPW), jnp.float32),
        pltpu.VMEM((EMBED, BPW), jnp.float32),
        pltpu.VMEM((BPW,), jnp.float32),
        pltpu.SemaphoreType.DMA,
    ],
)
def _sc_dots(x_hbm, y_hbm, cf_hbm, of_hbm, dots_hbm, xp, yp, cbuf, obuf, dv,
             sem):
    wid = lax.axis_index("s") * NC + lax.axis_index("c")
    base = wid * BPW
    pltpu.sync_copy(x_hbm.at[pl.ds(wid * NCH, NCH)], xp)
    pltpu.sync_copy(y_hbm.at[pl.ds(wid * NCH, NCH)], yp)
    for j in range(NCH):
        for k in range(CH // L):
            sl = pl.ds(k * L, L)
            vx = xp[j, sl]
            vy = yp[j, sl]
            xp[j, sl] = lax.shift_left(lax.shift_right_logical(vx, 7), 10) \
                + jnp.bitwise_and(vx, 127)
            yp[j, sl] = lax.shift_left(lax.shift_right_logical(vy, 7), 10) \
                + jnp.bitwise_and(vy, 127)

    for j in range(NCH):
        copies = []
        for c in range(EMBED):
            cbase = (c // 8) * GRP_WORDS + (c % 8) * 128
            clen = (NT - 1) * 1024 + 128
            copies.append(
                pltpu.async_copy(
                    cf_hbm.at[pl.ds(cbase, clen)].at[xp.at[j]],
                    cbuf.at[c, pl.ds(j * CH, CH)], sem))
            copies.append(
                pltpu.async_copy(
                    of_hbm.at[pl.ds(cbase, clen)].at[yp.at[j]],
                    obuf.at[c, pl.ds(j * CH, CH)], sem))
        for cp in copies:
            cp.wait()

    def body(g, carry):
        sl = pl.ds(g * L, L)
        acc = cbuf[0, sl] * obuf[0, sl]
        for c in range(1, EMBED):
            acc = acc + cbuf[c, sl] * obuf[c, sl]
        dv[sl] = acc
        return carry

    lax.fori_loop(0, BPW // L, body, 0)
    pltpu.sync_copy(dv, dots_hbm.at[pl.ds(base, BPW)])


def _tc_loss_body(d_ref, o_ref):
    d = d_ref[...]
    neg_abs = -jnp.abs(d)
    ls = jnp.minimum(d, 0.0) - jnp.log(1.0 + jnp.exp(neg_abs))
    o_ref[0, 0] = -jnp.sum(ls) / BATCH


_tc_loss = pl.pallas_call(
    _tc_loss_body,
    out_shape=jax.ShapeDtypeStruct((1, 1), jnp.float32),
    out_specs=pl.BlockSpec(memory_space=pltpu.SMEM),
)


def kernel(x, y, center_weight, out_weight):
    ct = center_weight.T
    ot = out_weight.T
    cf4, of4 = _tc_retile(ct, ot)
    cf = cf4.reshape(NGRP * NT * 8 * 128)
    of = of4.reshape(NGRP * NT * 8 * 128)
    x2 = x.reshape(NW * NCH, CH)
    y2 = y.reshape(NW * NCH, CH)
    dots = _sc_dots(x2, y2, cf, of)
    loss = _tc_loss(dots.reshape(BATCH // 128, 128))
    return loss[0, 0]


# speedup vs baseline: 4.7264x; 4.7264x over previous
import functools

import jax
import jax.numpy as jnp
from jax import lax
from jax.experimental import pallas as pl
from jax.experimental.pallas import tpu as pltpu
from jax.experimental.pallas import tpu_sc as plsc

VOCAB = 1000000
EMBED = 32
BATCH = 16384
NC, NS, L = 2, 16, 16
NW = NC * NS
BPW = BATCH // NW
CH = 128
NCH = BPW // CH

NGRP = 4
NT = 7813
GRP_WORDS = NT * 1024
TOTAL_TILES = NGRP * NT
WIN = 16
NWIN = (NT + WIN - 1) // WIN

_mesh = plsc.VectorSubcoreMesh(core_axis_name="c", subcore_axis_name="s")


RWIN = 64
RNB = (NT + RWIN - 1) // RWIN


def _tc_retile_body(c_in, o_in, c_out, o_out):
    for g in range(NGRP):
        for i in range(RWIN):
            sl = pl.ds(i * 128, 128)
            gsl = pl.ds(g * 8, 8)
            c_out[g, i] = c_in[gsl, sl]
            o_out[g, i] = o_in[gsl, sl]


_tc_retile = pl.pallas_call(
    _tc_retile_body,
    grid=(RNB,),
    in_specs=[
        pl.BlockSpec((32, RWIN * 128), lambda w: (0, w)),
        pl.BlockSpec((32, RWIN * 128), lambda w: (0, w)),
    ],
    out_specs=[
        pl.BlockSpec((NGRP, RWIN, 8, 128), lambda w: (0, w, 0, 0)),
        pl.BlockSpec((NGRP, RWIN, 8, 128), lambda w: (0, w, 0, 0)),
    ],
    out_shape=[
        jax.ShapeDtypeStruct((NGRP, NT, 8, 128), jnp.float32),
        jax.ShapeDtypeStruct((NGRP, NT, 8, 128), jnp.float32),
    ],
    compiler_params=pltpu.CompilerParams(
        dimension_semantics=("arbitrary",)),
)


@functools.partial(
    pl.kernel,
    out_type=jax.ShapeDtypeStruct((BATCH,), jnp.float32),
    mesh=_mesh,
    compiler_params=pltpu.CompilerParams(
        use_tc_tiling_on_sc=False, needs_layout_passes=False),
    scratch_types=[
        pltpu.VMEM((NCH, CH), jnp.int32),
        pltpu.VMEM((NCH, CH), jnp.int32),
        pltpu.VMEM((EMBED, BPW), jnp.float32),
        pltpu.VMEM((EMBED, BPW), jnp.float32),
        pltpu.VMEM((BPW,), jnp.float32),
        pltpu.SemaphoreType.DMA,
    ],
)
def _sc_dots(x_hbm, y_hbm, cf_hbm, of_hbm, dots_hbm, xp, yp, cbuf, obuf, dv,
             sem):
    wid = lax.axis_index("s") * NC + lax.axis_index("c")
    base = wid * BPW
    pltpu.sync_copy(x_hbm.at[pl.ds(wid * NCH, NCH)], xp)
    pltpu.sync_copy(y_hbm.at[pl.ds(wid * NCH, NCH)], yp)
    for j in range(NCH):
        for k in range(CH // L):
            sl = pl.ds(k * L, L)
            vx = xp[j, sl]
            vy = yp[j, sl]
            xp[j, sl] = lax.shift_left(lax.shift_right_logical(vx, 7), 10) \
                + jnp.bitwise_and(vx, 127)
            yp[j, sl] = lax.shift_left(lax.shift_right_logical(vy, 7), 10) \
                + jnp.bitwise_and(vy, 127)

    for j in range(NCH):
        copies = []
        for c in range(EMBED):
            cbase = (c // 8) * GRP_WORDS + (c % 8) * 128
            clen = (NT - 1) * 1024 + 128
            copies.append(
                pltpu.async_copy(
                    cf_hbm.at[pl.ds(cbase, clen)].at[xp.at[j]],
                    cbuf.at[c, pl.ds(j * CH, CH)], sem))
            copies.append(
                pltpu.async_copy(
                    of_hbm.at[pl.ds(cbase, clen)].at[yp.at[j]],
                    obuf.at[c, pl.ds(j * CH, CH)], sem))
        for cp in copies:
            cp.wait()

    def body(g, carry):
        sl = pl.ds(g * L, L)
        acc = cbuf[0, sl] * obuf[0, sl]
        for c in range(1, EMBED):
            acc = acc + cbuf[c, sl] * obuf[c, sl]
        dv[sl] = acc
        return carry

    lax.fori_loop(0, BPW // L, body, 0)
    pltpu.sync_copy(dv, dots_hbm.at[pl.ds(base, BPW)])


def _tc_loss_body(d_ref, o_ref):
    d = d_ref[...]
    neg_abs = -jnp.abs(d)
    ls = jnp.minimum(d, 0.0) - jnp.log(1.0 + jnp.exp(neg_abs))
    o_ref[0, 0] = -jnp.sum(ls) / BATCH


_tc_loss = pl.pallas_call(
    _tc_loss_body,
    out_shape=jax.ShapeDtypeStruct((1, 1), jnp.float32),
    out_specs=pl.BlockSpec(memory_space=pltpu.SMEM),
)


def kernel(x, y, center_weight, out_weight):
    ct = center_weight.T
    ot = out_weight.T
    cf4, of4 = _tc_retile(ct, ot)
    cf = cf4.reshape(NGRP * NT * 8 * 128)
    of = of4.reshape(NGRP * NT * 8 * 128)
    x2 = x.reshape(NW * NCH, CH)
    y2 = y.reshape(NW * NCH, CH)
    dots = _sc_dots(x2, y2, cf, of)
    loss = _tc_loss(dots.reshape(BATCH // 128, 128))
    return loss[0, 0]
